# baseline (speedup 1.0000x reference)
import functools

import jax
import jax.numpy as jnp
from jax import lax
from jax.experimental import pallas as pl
from jax.experimental.pallas import tpu as pltpu
from jax.experimental.pallas import tpu_sc as plsc

NC = 2
NS = 16
NW = NC * NS
L = 16
N = 8192
QPW = N * 4 // NW
QG = 16
NVEC = N // L

_mesh = plsc.VectorSubcoreMesh(
    core_axis_name="c", subcore_axis_name="s", num_cores=NC, num_subcores=NS
)


def _bf16_round(v):
    u = plsc.bitcast(v, jnp.uint32)
    lsb = (u >> jnp.uint32(16)) & jnp.uint32(1)
    r = (u + jnp.uint32(32767) + lsb) & jnp.uint32(0xFFFF0000)
    return plsc.bitcast(r, jnp.float32)


@functools.partial(
    pl.kernel,
    out_type=jax.ShapeDtypeStruct((NW, L), jnp.float32),
    mesh=_mesh,
    compiler_params=pltpu.CompilerParams(needs_layout_passes=False),
    scratch_types=[
        pltpu.VMEM((N,), jnp.float32),
        pltpu.VMEM((N,), jnp.float32),
        pltpu.VMEM((N,), jnp.float32),
        pltpu.VMEM((N,), jnp.float32),
        pltpu.VMEM((N,), jnp.float32),
        pltpu.VMEM((N,), jnp.float32),
        pltpu.VMEM((N,), jnp.float32),
        pltpu.VMEM((QPW,), jnp.float32),
        pltpu.VMEM((QPW,), jnp.float32),
        pltpu.VMEM((QPW,), jnp.float32),
        pltpu.VMEM((L,), jnp.float32),
    ],
)
def _chamfer_sc(xs, ys, zs, out, cx, cy, cz, rx, ry, rz, cc, qx, qy, qz, res):
    cid = lax.axis_index("c")
    sid = lax.axis_index("s")
    wid = sid * NC + cid
    sweep = wid // 8
    chunk = wid % 8
    direction = sweep // 2
    batch = sweep % 2
    qrow = direction * 2 + batch
    crow = (1 - direction) * 2 + batch
    qbase = chunk * QPW

    pltpu.sync_copy(xs.at[crow], cx)
    pltpu.sync_copy(ys.at[crow], cy)
    pltpu.sync_copy(zs.at[crow], cz)
    pltpu.sync_copy(xs.at[qrow, pl.ds(qbase, QPW)], qx)
    pltpu.sync_copy(ys.at[qrow, pl.ds(qbase, QPW)], qy)
    pltpu.sync_copy(zs.at[qrow, pl.ds(qbase, QPW)], qz)

    def pre_body(j, carry):
        sl = pl.ds(j * L, L)
        x = cx[sl]
        y = cy[sl]
        z = cz[sl]
        cc[sl] = x * x + y * y + z * z
        rx[sl] = _bf16_round(x)
        ry[sl] = _bf16_round(y)
        rz[sl] = _bf16_round(z)
        return carry

    lax.fori_loop(0, NVEC, pre_body, 0)

    big = jnp.full((L,), 3.0e38, jnp.float32)
    izero = jnp.zeros((L,), jnp.int32)
    lane = lax.iota(jnp.int32, L)
    imax = jnp.int32(2147483647)

    def group_body(g, accd):
        qb = g * QG
        vx = qx[pl.ds(qb, QG)]
        vy = qy[pl.ds(qb, QG)]
        vz = qz[pl.ds(qb, QG)]
        mwx = -2.0 * _bf16_round(vx)
        mwy = -2.0 * _bf16_round(vy)
        mwz = -2.0 * _bf16_round(vz)
        m2x = [mwx[i] for i in range(QG)]
        m2y = [mwy[i] for i in range(QG)]
        m2z = [mwz[i] for i in range(QG)]

        def inner(j, carry):
            mins, idxs, jv = carry
            sl = pl.ds(j * L, L)
            x = rx[sl]
            y = ry[sl]
            z = rz[sl]
            base = cc[sl]
            nmins = []
            nidxs = []
            for i in range(QG):
                t = base + x * m2x[i] + y * m2y[i] + z * m2z[i]
                better = t < mins[i]
                nmins.append(jnp.where(better, t, mins[i]))
                nidxs.append(jnp.where(better, jv, idxs[i]))
            return tuple(nmins), tuple(nidxs), jv + L

        mins, idxs, _ = lax.fori_loop(
            0, NVEC, inner, ((big,) * QG, (izero,) * QG, lane)
        )
        idxvec = izero
        for i in range(QG):
            hv = -plsc.cummax(-mins[i])
            masked = jnp.where(mins[i] == hv[QG - 1], idxs[i], imax)
            win = -plsc.cummax(-masked)
            idxvec = jnp.where(lane == i, win[QG - 1], idxvec)
        gx = plsc.load_gather(cx, [idxvec])
        gy = plsc.load_gather(cy, [idxvec])
        gz = plsc.load_gather(cz, [idxvec])
        dx = vx - gx
        dy = vy - gy
        dz = vz - gz
        return accd + (dx * dx + dy * dy + dz * dz)

    accd = lax.fori_loop(0, QPW // QG, group_body, jnp.zeros((L,), jnp.float32))
    res[...] = accd
    pltpu.sync_copy(res, out.at[wid])


def kernel(pos, x_hat):
    xs = jnp.concatenate([pos[:, :, 0], x_hat[:, :, 0]], axis=0)
    ys = jnp.concatenate([pos[:, :, 1], x_hat[:, :, 1]], axis=0)
    zs = jnp.concatenate([pos[:, :, 2], x_hat[:, :, 2]], axis=0)
    partials = _chamfer_sc(xs, ys, zs)
    return jnp.sum(partials) * jnp.float32(1.0 / (2.0 * N))


# speedup vs baseline: 2.6281x; 2.6281x over previous
import functools

import jax
import jax.numpy as jnp
from jax import lax
from jax.experimental import pallas as pl
from jax.experimental.pallas import tpu as pltpu
from jax.experimental.pallas import tpu_sc as plsc

NC = 2
NS = 16
NW = NC * NS
L = 16
N = 8192
QPW = N * 4 // NW
QG = 8
NVEC = N // L

_mesh = plsc.VectorSubcoreMesh(
    core_axis_name="c", subcore_axis_name="s", num_cores=NC, num_subcores=NS
)


def _bf16_round(v):
    u = plsc.bitcast(v, jnp.uint32)
    lsb = (u >> jnp.uint32(16)) & jnp.uint32(1)
    r = (u + jnp.uint32(32767) + lsb) & jnp.uint32(0xFFFF0000)
    return plsc.bitcast(r, jnp.float32)


@functools.partial(
    pl.kernel,
    out_type=jax.ShapeDtypeStruct((NW, L), jnp.float32),
    mesh=_mesh,
    compiler_params=pltpu.CompilerParams(needs_layout_passes=False),
    scratch_types=[
        pltpu.VMEM((N,), jnp.float32),
        pltpu.VMEM((N,), jnp.float32),
        pltpu.VMEM((N,), jnp.float32),
        pltpu.VMEM((N,), jnp.float32),
        pltpu.VMEM((N,), jnp.float32),
        pltpu.VMEM((N,), jnp.float32),
        pltpu.VMEM((N,), jnp.float32),
        pltpu.VMEM((QPW,), jnp.float32),
        pltpu.VMEM((QPW,), jnp.float32),
        pltpu.VMEM((QPW,), jnp.float32),
        pltpu.VMEM((L,), jnp.float32),
    ],
)
def _chamfer_sc(xs, ys, zs, out, cx, cy, cz, rx, ry, rz, cc, qx, qy, qz, res):
    cid = lax.axis_index("c")
    sid = lax.axis_index("s")
    wid = sid * NC + cid
    sweep = wid // 8
    chunk = wid % 8
    direction = sweep // 2
    batch = sweep % 2
    qrow = direction * 2 + batch
    crow = (1 - direction) * 2 + batch
    qbase = chunk * QPW

    pltpu.sync_copy(xs.at[crow], cx)
    pltpu.sync_copy(ys.at[crow], cy)
    pltpu.sync_copy(zs.at[crow], cz)
    pltpu.sync_copy(xs.at[qrow, pl.ds(qbase, QPW)], qx)
    pltpu.sync_copy(ys.at[qrow, pl.ds(qbase, QPW)], qy)
    pltpu.sync_copy(zs.at[qrow, pl.ds(qbase, QPW)], qz)

    def pre_body(j, carry):
        sl = pl.ds(j * L, L)
        x = cx[sl]
        y = cy[sl]
        z = cz[sl]
        cc[sl] = x * x + y * y + z * z
        rx[sl] = _bf16_round(x)
        ry[sl] = _bf16_round(y)
        rz[sl] = _bf16_round(z)
        return carry

    lax.fori_loop(0, NVEC, pre_body, 0)

    big = jnp.full((L,), 3.0e38, jnp.float32)
    izero = jnp.zeros((L,), jnp.int32)
    lane = lax.iota(jnp.int32, L)
    imax = jnp.int32(2147483647)

    def group_body(g, accd):
        qb = g * L
        vx = qx[pl.ds(qb, L)]
        vy = qy[pl.ds(qb, L)]
        vz = qz[pl.ds(qb, L)]
        mwx = -2.0 * _bf16_round(vx)
        mwy = -2.0 * _bf16_round(vy)
        mwz = -2.0 * _bf16_round(vz)

        idxvec = izero
        for half in range(L // QG):
            lo = half * QG
            m2x = [mwx[lo + i] for i in range(QG)]
            m2y = [mwy[lo + i] for i in range(QG)]
            m2z = [mwz[lo + i] for i in range(QG)]

            def inner(j, carry):
                mins, idxs, jv = carry
                sl = pl.ds(j * L, L)
                x = rx[sl]
                y = ry[sl]
                z = rz[sl]
                base = cc[sl]
                nmins = []
                nidxs = []
                for i in range(QG):
                    t = base + x * m2x[i] + y * m2y[i] + z * m2z[i]
                    better = t < mins[i]
                    nmins.append(jnp.where(better, t, mins[i]))
                    nidxs.append(jnp.where(better, jv, idxs[i]))
                return tuple(nmins), tuple(nidxs), jv + L

            mins, idxs, _ = lax.fori_loop(
                0, NVEC, inner, ((big,) * QG, (izero,) * QG, lane), unroll=2
            )
            for i in range(QG):
                hv = -plsc.cummax(-mins[i])
                masked = jnp.where(mins[i] == hv[L - 1], idxs[i], imax)
                win = -plsc.cummax(-masked)
                idxvec = jnp.where(lane == lo + i, win[L - 1], idxvec)
        gx = plsc.load_gather(cx, [idxvec])
        gy = plsc.load_gather(cy, [idxvec])
        gz = plsc.load_gather(cz, [idxvec])
        dx = vx - gx
        dy = vy - gy
        dz = vz - gz
        return accd + (dx * dx + dy * dy + dz * dz)

    accd = lax.fori_loop(0, QPW // L, group_body, jnp.zeros((L,), jnp.float32))
    res[...] = accd
    pltpu.sync_copy(res, out.at[wid])


def kernel(pos, x_hat):
    xs = jnp.concatenate([pos[:, :, 0], x_hat[:, :, 0]], axis=0)
    ys = jnp.concatenate([pos[:, :, 1], x_hat[:, :, 1]], axis=0)
    zs = jnp.concatenate([pos[:, :, 2], x_hat[:, :, 2]], axis=0)
    partials = _chamfer_sc(xs, ys, zs)
    return jnp.sum(partials) * jnp.float32(1.0 / (2.0 * N))


# speedup vs baseline: 5.1031x; 1.9417x over previous
import functools

import jax
import jax.numpy as jnp
from jax import lax
from jax.experimental import pallas as pl
from jax.experimental.pallas import tpu as pltpu
from jax.experimental.pallas import tpu_sc as plsc

NC = 2
NS = 16
NW = NC * NS
L = 16
N = 8192
NSC_SWEEPS = 2
WPS = NW // NSC_SWEEPS
QPW = N // WPS
QG = 8
NVEC = N // L

_mesh = plsc.VectorSubcoreMesh(
    core_axis_name="c", subcore_axis_name="s", num_cores=NC, num_subcores=NS
)


def _bf16_round(v):
    u = plsc.bitcast(v, jnp.uint32)
    lsb = (u >> jnp.uint32(16)) & jnp.uint32(1)
    r = (u + jnp.uint32(32767) + lsb) & jnp.uint32(0xFFFF0000)
    return plsc.bitcast(r, jnp.float32)


@functools.partial(
    pl.kernel,
    out_type=jax.ShapeDtypeStruct((NW, L), jnp.float32),
    mesh=_mesh,
    compiler_params=pltpu.CompilerParams(needs_layout_passes=False),
    scratch_types=[
        pltpu.VMEM((N,), jnp.float32),
        pltpu.VMEM((N,), jnp.float32),
        pltpu.VMEM((N,), jnp.float32),
        pltpu.VMEM((N,), jnp.float32),
        pltpu.VMEM((N,), jnp.float32),
        pltpu.VMEM((N,), jnp.float32),
        pltpu.VMEM((N,), jnp.float32),
        pltpu.VMEM((QPW,), jnp.float32),
        pltpu.VMEM((QPW,), jnp.float32),
        pltpu.VMEM((QPW,), jnp.float32),
        pltpu.VMEM((L,), jnp.float32),
    ],
)
def _chamfer_sc(xs, ys, zs, out, cx, cy, cz, rx, ry, rz, cc, qx, qy, qz, res):
    cid = lax.axis_index("c")
    sid = lax.axis_index("s")
    wid = sid * NC + cid
    batch = wid // WPS
    chunk = wid % WPS
    qrow = batch
    crow = 2 + batch
    qbase = chunk * QPW

    pltpu.sync_copy(xs.at[crow], cx)
    pltpu.sync_copy(ys.at[crow], cy)
    pltpu.sync_copy(zs.at[crow], cz)
    pltpu.sync_copy(xs.at[qrow, pl.ds(qbase, QPW)], qx)
    pltpu.sync_copy(ys.at[qrow, pl.ds(qbase, QPW)], qy)
    pltpu.sync_copy(zs.at[qrow, pl.ds(qbase, QPW)], qz)

    def pre_body(j, carry):
        sl = pl.ds(j * L, L)
        x = cx[sl]
        y = cy[sl]
        z = cz[sl]
        cc[sl] = x * x + y * y + z * z
        rx[sl] = _bf16_round(x)
        ry[sl] = _bf16_round(y)
        rz[sl] = _bf16_round(z)
        return carry

    lax.fori_loop(0, NVEC, pre_body, 0)

    big = jnp.full((L,), 3.0e38, jnp.float32)
    izero = jnp.zeros((L,), jnp.int32)
    lane = lax.iota(jnp.int32, L)
    imax = jnp.int32(2147483647)

    def group_body(g, accd):
        qb = g * L
        vx = qx[pl.ds(qb, L)]
        vy = qy[pl.ds(qb, L)]
        vz = qz[pl.ds(qb, L)]
        mwx = -2.0 * _bf16_round(vx)
        mwy = -2.0 * _bf16_round(vy)
        mwz = -2.0 * _bf16_round(vz)

        idxvec = izero
        for half in range(L // QG):
            lo = half * QG
            m2x = [mwx[lo + i] for i in range(QG)]
            m2y = [mwy[lo + i] for i in range(QG)]
            m2z = [mwz[lo + i] for i in range(QG)]

            def inner(j, carry):
                mins, idxs, jv = carry
                sl = pl.ds(j * L, L)
                x = rx[sl]
                y = ry[sl]
                z = rz[sl]
                base = cc[sl]
                nmins = []
                nidxs = []
                for i in range(QG):
                    t = base + x * m2x[i] + y * m2y[i] + z * m2z[i]
                    better = t < mins[i]
                    nmins.append(jnp.where(better, t, mins[i]))
                    nidxs.append(jnp.where(better, jv, idxs[i]))
                return tuple(nmins), tuple(nidxs), jv + L

            mins, idxs, _ = lax.fori_loop(
                0, NVEC, inner, ((big,) * QG, (izero,) * QG, lane), unroll=2
            )
            for i in range(QG):
                hv = -plsc.cummax(-mins[i])
                masked = jnp.where(mins[i] == hv[L - 1], idxs[i], imax)
                win = -plsc.cummax(-masked)
                idxvec = jnp.where(lane == lo + i, win[L - 1], idxvec)
        gx = plsc.load_gather(cx, [idxvec])
        gy = plsc.load_gather(cy, [idxvec])
        gz = plsc.load_gather(cz, [idxvec])
        dx = vx - gx
        dy = vy - gy
        dz = vz - gz
        return accd + (dx * dx + dy * dy + dz * dz)

    accd = lax.fori_loop(0, QPW // L, group_body, jnp.zeros((L,), jnp.float32))
    res[...] = accd
    pltpu.sync_copy(res, out.at[wid])


RT = 256
TSTEPS = 2 * N // RT


def _tc_body(q8_ref, qb_ref, c8_ref, cb_ref, out_ref):
    step = pl.program_id(0)
    q8 = q8_ref[...]
    qb = qb_ref[...]
    c8 = c8_ref[0]
    cb = cb_ref[0]
    aa = jnp.sum(q8 * q8, axis=0)
    bb = jnp.sum(c8 * c8, axis=0)
    ab = lax.dot_general(
        qb, cb, (((0,), (0,)), ((), ())),
        preferred_element_type=jnp.float32,
    )
    score = bb[None, :] - 2.0 * ab
    abf = lax.dot_general(
        q8, c8, (((0,), (0,)), ((), ())),
        precision=lax.Precision.HIGHEST,
        preferred_element_type=jnp.float32,
    )
    exact = (aa[:, None] + bb[None, :]) - 2.0 * abf
    rowmin = jnp.min(score, axis=1)
    bigf = jnp.float32(3.0e38)
    dwin = jnp.min(jnp.where(score == rowmin[:, None], exact, bigf), axis=1)
    part = jnp.reshape(dwin, (RT // 128, 128))

    @pl.when(step == 0)
    def _init():
        out_ref[...] = jnp.zeros_like(out_ref)

    out_ref[...] += jnp.sum(part, axis=0, keepdims=True)


_tc_call = pl.pallas_call(
    _tc_body,
    grid=(TSTEPS,),
    in_specs=[
        pl.BlockSpec((8, RT), lambda i: (0, i)),
        pl.BlockSpec((8, RT), lambda i: (0, i)),
        pl.BlockSpec((1, 8, N), lambda i: (i * RT // N, 0, 0)),
        pl.BlockSpec((1, 8, N), lambda i: (i * RT // N, 0, 0)),
    ],
    out_specs=pl.BlockSpec((1, 128), lambda i: (0, 0)),
    out_shape=jax.ShapeDtypeStruct((1, 128), jnp.float32),
    compiler_params=pltpu.CompilerParams(
        dimension_semantics=("arbitrary",),
    ),
)


def kernel(pos, x_hat):
    xs = jnp.concatenate([pos[:, :, 0], x_hat[:, :, 0]], axis=0)
    ys = jnp.concatenate([pos[:, :, 1], x_hat[:, :, 1]], axis=0)
    zs = jnp.concatenate([pos[:, :, 2], x_hat[:, :, 2]], axis=0)
    sc_partials = _chamfer_sc(xs, ys, zs)

    q = jnp.concatenate([x_hat[0], x_hat[1]], axis=0)
    q8 = jnp.pad(q.T, ((0, 5), (0, 0)))
    c8 = jnp.pad(jnp.transpose(pos, (0, 2, 1)), ((0, 0), (0, 5), (0, 0)))
    qb = q8.astype(jnp.bfloat16)
    cb = c8.astype(jnp.bfloat16)
    tc_part = _tc_call(q8, qb, c8, cb)

    total = jnp.sum(sc_partials) + jnp.sum(tc_part)
    return total * jnp.float32(1.0 / (2.0 * N))
